# Initial kernel scaffold; baseline (speedup 1.0000x reference)
import jax
import jax.numpy as jnp
from jax.experimental import pallas as pl


def kernel(x, unused, e, W, b, Wa, rms_w, rms_b):
    raise NotImplementedError("write your pallas kernel here")


# speedup vs baseline: 45.6358x; 45.6358x over previous
import functools

import jax
import jax.numpy as jnp
from jax import lax
from jax.experimental import pallas as pl
from jax.experimental.pallas import tpu as pltpu
from jax.experimental.pallas import tpu_sc as plsc

_N = 10000
_E = 320000
_H = 128
_EPS = 1e-5

_NC = 2
_NS = 16
_L = 16
_NW = _NC * _NS
_EPT = _E // _NW
_CHUNKS = _EPT // _L


def _tc_logits_body(x_ref, w_ref, wa2_ref, b_ref, pq_ref):
    wa2 = wa2_ref[...]
    uv = jax.lax.dot_general(wa2, w_ref[...], (((1,), (0,)), ((), ())),
                             preferred_element_type=jnp.float32)
    pq = jax.lax.dot_general(uv, x_ref[...], (((1,), (1,)), ((), ())),
                             preferred_element_type=jnp.float32)
    c = jnp.sum((wa2[0:1, :] + wa2[1:2, :]) * b_ref[...])
    row = jax.lax.broadcasted_iota(jnp.int32, (2, _N), 0)
    pq_ref[...] = pq + jnp.where(row == 1, c, jnp.float32(0.0))


def _sc_edge_body(pq_hbm, e0_hbm, e1_hbm, out_hbm, p_v, q_v, e0_v, e1_v, n_v):
    cid = lax.axis_index("c")
    sid = lax.axis_index("s")
    wid = sid * _NC + cid
    base = pl.multiple_of(wid * _EPT, 8)

    pltpu.sync_copy(pq_hbm.at[0], p_v)
    pltpu.sync_copy(pq_hbm.at[1], q_v)
    pltpu.sync_copy(e0_hbm.at[pl.ds(base, _EPT)], e0_v)
    pltpu.sync_copy(e1_hbm.at[pl.ds(base, _EPT)], e1_v)

    def _zero(i, _):
        n_v[pl.ds(pl.multiple_of(i * _L, _L), _L)] = jnp.zeros((_L,), jnp.float32)
        return _

    lax.fori_loop(0, _CHUNKS, _zero, None)

    def _step(i, _):
        off = pl.multiple_of(i * _L, _L)
        i0 = e0_v[pl.ds(off, _L)]
        i1 = e1_v[pl.ds(off, _L)]
        p = plsc.load_gather(p_v, [i0])
        q = plsc.load_gather(q_v, [i1])
        z = p + q
        a = jnp.exp(jnp.where(z > 0.0, z, 0.2 * z))
        plsc.addupdate_scatter(n_v, [i1], a)
        return _

    lax.fori_loop(0, _CHUNKS, _step, None)

    pltpu.sync_copy(n_v, out_hbm.at[wid])


def _tc_norm_body(x_ref, np_ref, w_ref, b_ref, o_ref):
    x = x_ref[...]
    n = jnp.sum(np_ref[...], axis=1, keepdims=True)
    den = jnp.where(n == 0.0, jnp.float32(1.0), n)
    h = jnp.maximum(x / den, 0.0) + x
    inv = jax.lax.rsqrt(jnp.mean(h * h, axis=1, keepdims=True) + _EPS)
    o_ref[...] = h * inv * w_ref[...] + b_ref[...]


@functools.partial(jax.jit, static_argnums=())
def _edge_nsum(pq, e0, e1):
    mesh = plsc.VectorSubcoreMesh(core_axis_name="c", subcore_axis_name="s",
                                  num_cores=_NC, num_subcores=_NS)
    return pl.kernel(
        _sc_edge_body,
        out_type=jax.ShapeDtypeStruct((_NW, _N), jnp.float32),
        mesh=mesh,
        compiler_params=pltpu.CompilerParams(needs_layout_passes=False),
        scratch_types=[
            pltpu.VMEM((_N,), jnp.float32),
            pltpu.VMEM((_N,), jnp.float32),
            pltpu.VMEM((_EPT,), jnp.int32),
            pltpu.VMEM((_EPT,), jnp.int32),
            pltpu.VMEM((_N,), jnp.float32),
        ],
    )(pq, e0, e1)


def kernel(x, unused, e, W, b, Wa, rms_w, rms_b):
    x = x.astype(jnp.float32)
    e = e.astype(jnp.int32)
    e0 = e[:, 0]
    e1 = e[:, 1]

    pq = pl.pallas_call(
        _tc_logits_body,
        out_shape=jax.ShapeDtypeStruct((2, _N), jnp.float32),
    )(x, W, Wa.reshape(2, _H), b.reshape(1, _H))

    nparts = _edge_nsum(pq, e0, e1)

    out = pl.pallas_call(
        _tc_norm_body,
        out_shape=jax.ShapeDtypeStruct((_N, _H), jnp.float32),
    )(x, nparts.T, rms_w.reshape(1, _H), rms_b.reshape(1, _H))
    return out


# speedup vs baseline: 46.6977x; 1.0233x over previous
import functools

import jax
import jax.numpy as jnp
from jax import lax
from jax.experimental import pallas as pl
from jax.experimental.pallas import tpu as pltpu
from jax.experimental.pallas import tpu_sc as plsc

_N = 10000
_E = 320000
_H = 128
_EPS = 1e-5

_NC = 2
_NS = 16
_L = 16
_NW = _NC * _NS
_EPT = _E // _NW
_CHUNKS = _EPT // _L
_UNROLL = 25


def _tc_logits_body(x_ref, w_ref, wa2_ref, b_ref, pq_ref):
    wa2 = wa2_ref[...]
    uv = jax.lax.dot_general(wa2, w_ref[...], (((1,), (0,)), ((), ())),
                             preferred_element_type=jnp.float32)
    pq = jax.lax.dot_general(uv, x_ref[...], (((1,), (1,)), ((), ())),
                             preferred_element_type=jnp.float32)
    c = jnp.sum((wa2[0:1, :] + wa2[1:2, :]) * b_ref[...])
    row = jax.lax.broadcasted_iota(jnp.int32, (2, _N), 0)
    pq_ref[...] = pq + jnp.where(row == 1, c, jnp.float32(0.0))


def _sc_edge_body(pq_hbm, e0_hbm, e1_hbm, out_hbm, p_v, q_v, e0_v, e1_v, n_v):
    cid = lax.axis_index("c")
    sid = lax.axis_index("s")
    wid = sid * _NC + cid
    base = pl.multiple_of(wid * _EPT, 8)

    pltpu.sync_copy(pq_hbm.at[0], p_v)
    pltpu.sync_copy(pq_hbm.at[1], q_v)
    pltpu.sync_copy(e0_hbm.at[pl.ds(base, _EPT)], e0_v)
    pltpu.sync_copy(e1_hbm.at[pl.ds(base, _EPT)], e1_v)

    def _zero(i, _):
        base = pl.multiple_of(i * (_L * _UNROLL), _L)
        for j in range(_UNROLL):
            n_v[pl.ds(base + j * _L, _L)] = jnp.zeros((_L,), jnp.float32)
        return _

    lax.fori_loop(0, _CHUNKS // _UNROLL, _zero, None)

    def _step(i, _):
        base = pl.multiple_of(i * (_L * _UNROLL), _L)
        for j in range(_UNROLL):
            off = base + j * _L
            i0 = e0_v[pl.ds(off, _L)]
            i1 = e1_v[pl.ds(off, _L)]
            p = plsc.load_gather(p_v, [i0])
            q = plsc.load_gather(q_v, [i1])
            z = p + q
            a = jnp.exp(jnp.where(z > 0.0, z, 0.2 * z))
            plsc.addupdate_scatter(n_v, [i1], a)
        return _

    lax.fori_loop(0, _CHUNKS // _UNROLL, _step, None)

    pltpu.sync_copy(n_v, out_hbm.at[wid])


def _tc_norm_body(x_ref, np_ref, w_ref, b_ref, o_ref):
    x = x_ref[...]
    n = jnp.sum(np_ref[...], axis=1, keepdims=True)
    den = jnp.where(n == 0.0, jnp.float32(1.0), n)
    h = jnp.maximum(x / den, 0.0) + x
    inv = jax.lax.rsqrt(jnp.mean(h * h, axis=1, keepdims=True) + _EPS)
    o_ref[...] = h * inv * w_ref[...] + b_ref[...]


@functools.partial(jax.jit, static_argnums=())
def _edge_nsum(pq, e0, e1):
    mesh = plsc.VectorSubcoreMesh(core_axis_name="c", subcore_axis_name="s",
                                  num_cores=_NC, num_subcores=_NS)
    return pl.kernel(
        _sc_edge_body,
        out_type=jax.ShapeDtypeStruct((_NW, _N), jnp.float32),
        mesh=mesh,
        compiler_params=pltpu.CompilerParams(needs_layout_passes=False),
        scratch_types=[
            pltpu.VMEM((_N,), jnp.float32),
            pltpu.VMEM((_N,), jnp.float32),
            pltpu.VMEM((_EPT,), jnp.int32),
            pltpu.VMEM((_EPT,), jnp.int32),
            pltpu.VMEM((_N,), jnp.float32),
        ],
    )(pq, e0, e1)


def kernel(x, unused, e, W, b, Wa, rms_w, rms_b):
    x = x.astype(jnp.float32)
    e = e.astype(jnp.int32)
    e0 = e[:, 0]
    e1 = e[:, 1]

    pq = pl.pallas_call(
        _tc_logits_body,
        out_shape=jax.ShapeDtypeStruct((2, _N), jnp.float32),
    )(x, W, Wa.reshape(2, _H), b.reshape(1, _H))

    nparts = _edge_nsum(pq, e0, e1)

    out = pl.pallas_call(
        _tc_norm_body,
        out_shape=jax.ShapeDtypeStruct((_N, _H), jnp.float32),
    )(x, nparts.T, rms_w.reshape(1, _H), rms_b.reshape(1, _H))
    return out
